# Initial kernel scaffold; baseline (speedup 1.0000x reference)
import jax
import jax.numpy as jnp
from jax.experimental import pallas as pl


def kernel(class_logits, box_regression, proposals):
    raise NotImplementedError("write your pallas kernel here")


# speedup vs baseline: 6.3253x; 6.3253x over previous
import jax
import jax.numpy as jnp
from jax import lax
from jax.experimental import pallas as pl
from jax.experimental.pallas import tpu as pltpu
import math

_N = 20000
_C = 21
_IMG = 800.0
_SCORE_THRESH = 0.05
_NMS_THRESH = 0.5
_DETS = 100
_CLIP = math.log(1000.0 / 16.0)
_NEG = -1e10


def _nms_body(lt_ref, dx_ref, dy_ref, dw_ref, dh_ref, pt_ref,
              ob_ref, os_ref, ol_ref):
    lt = lt_ref[...]
    m = jnp.max(lt, axis=0, keepdims=True)
    e = jnp.exp(lt - m)
    p = e / jnp.sum(e, axis=0, keepdims=True)

    x1r = pt_ref[0:1, :]
    y1r = pt_ref[1:2, :]
    x2r = pt_ref[2:3, :]
    y2r = pt_ref[3:4, :]
    wid = x2r - x1r
    hei = y2r - y1r
    cx = x1r + 0.5 * wid
    cy = y1r + 0.5 * hei
    dx = dx_ref[...] / 10.0
    dy = dy_ref[...] / 10.0
    dw = jnp.minimum(dw_ref[...] / 5.0, _CLIP)
    dh = jnp.minimum(dh_ref[...] / 5.0, _CLIP)
    pcx = dx * wid + cx
    pcy = dy * hei + cy
    pw = jnp.exp(dw) * wid
    ph = jnp.exp(dh) * hei
    x1 = jnp.clip(pcx - 0.5 * pw, 0.0, _IMG)
    y1 = jnp.clip(pcy - 0.5 * ph, 0.0, _IMG)
    x2 = jnp.clip(pcx + 0.5 * pw, 0.0, _IMG)
    y2 = jnp.clip(pcy + 0.5 * ph, 0.0, _IMG)

    rio = lax.broadcasted_iota(jnp.int32, (_C, _N), 0)
    lio = lax.broadcasted_iota(jnp.int32, (_C, _N), 1)

    ws = x2 - x1
    hs = y2 - y1
    valid = (rio > 0) & (p > _SCORE_THRESH) & (ws >= 0.01) & (hs >= 0.01)
    cur0 = jnp.where(valid, p, _NEG)

    big = jnp.int32(1 << 30)
    l4 = lax.broadcasted_iota(jnp.int32, (1, 4), 1)

    def body(i, cur):
        gmax = jnp.max(cur)
        n = jnp.min(jnp.where(cur == gmax, lio, big))
        c = jnp.min(jnp.where((cur == gmax) & (lio == n), rio, big))
        pm = (rio == c) & (lio == n)
        px1 = jnp.max(jnp.where(pm, x1, -jnp.inf))
        py1 = jnp.max(jnp.where(pm, y1, -jnp.inf))
        px2 = jnp.max(jnp.where(pm, x2, -jnp.inf))
        py2 = jnp.max(jnp.where(pm, y2, -jnp.inf))
        ix1 = jnp.maximum(px1, x1)
        iy1 = jnp.maximum(py1, y1)
        ix2 = jnp.minimum(px2, x2)
        iy2 = jnp.minimum(py2, y2)
        inter = jnp.maximum(ix2 - ix1, 0.0) * jnp.maximum(iy2 - iy1, 0.0)
        a1 = (px2 - px1) * (py2 - py1)
        a2 = (x2 - x1) * (y2 - y1)
        iou = inter / (a1 + a2 - inter + 1e-9)
        cur = jnp.where((rio == c) & (iou > _NMS_THRESH), _NEG, cur)
        cur = jnp.where(pm, _NEG, cur)
        ok = gmax > (_NEG / 2)
        okf = jnp.where(ok, 1.0, 0.0)
        row = jnp.where(l4 == 0, px1,
              jnp.where(l4 == 1, py1,
              jnp.where(l4 == 2, px2, py2))) * okf
        ob_ref[pl.ds(i, 1), :] = row
        os_ref[pl.ds(i, 1), :] = jnp.full((1, 1), jnp.where(ok, gmax, 0.0),
                                          dtype=jnp.float32)
        ol_ref[pl.ds(i, 1), :] = jnp.full((1, 1), jnp.where(ok, c, 0),
                                          dtype=jnp.int32)
        return cur

    lax.fori_loop(0, _DETS, body, cur0)


def kernel(class_logits, box_regression, proposals):
    lt = class_logits.T
    br = box_regression.reshape(_N, _C, 4)
    dxt = br[:, :, 0].T
    dyt = br[:, :, 1].T
    dwt = br[:, :, 2].T
    dht = br[:, :, 3].T
    pt = proposals.T
    ob, osc, olb = pl.pallas_call(
        _nms_body,
        out_shape=(
            jax.ShapeDtypeStruct((_DETS, 4), jnp.float32),
            jax.ShapeDtypeStruct((_DETS, 1), jnp.float32),
            jax.ShapeDtypeStruct((_DETS, 1), jnp.int32),
        ),
        compiler_params=pltpu.CompilerParams(
            vmem_limit_bytes=128 * 1024 * 1024,
        ),
    )(lt, dxt, dyt, dwt, dht, pt)
    return ob, osc.reshape(_DETS), olb.reshape(_DETS)


# speedup vs baseline: 18.1795x; 2.8741x over previous
import jax
import jax.numpy as jnp
from jax import lax
from jax.experimental import pallas as pl
from jax.experimental.pallas import tpu as pltpu
import math

_N = 20000
_C = 21
_IMG = 800.0
_SCORE_THRESH = 0.05
_NMS_THRESH = 0.5
_DETS = 100
_CLIP = math.log(1000.0 / 16.0)
_NEG = -1e10


def _nms_body(lt_ref, dx_ref, dy_ref, dw_ref, dh_ref, pt_ref,
              ob_ref, os_ref, ol_ref,
              cur_ref, cmx_ref, x1_ref, y1_ref, x2_ref, y2_ref):
    lt = lt_ref[...]
    m = jnp.max(lt, axis=0, keepdims=True)
    e = jnp.exp(lt - m)
    p = e / jnp.sum(e, axis=0, keepdims=True)

    x1r = pt_ref[0:1, :]
    y1r = pt_ref[1:2, :]
    x2r = pt_ref[2:3, :]
    y2r = pt_ref[3:4, :]
    wid = x2r - x1r
    hei = y2r - y1r
    cx = x1r + 0.5 * wid
    cy = y1r + 0.5 * hei
    dx = dx_ref[...] / 10.0
    dy = dy_ref[...] / 10.0
    dw = jnp.minimum(dw_ref[...] / 5.0, _CLIP)
    dh = jnp.minimum(dh_ref[...] / 5.0, _CLIP)
    pcx = dx * wid + cx
    pcy = dy * hei + cy
    pw = jnp.exp(dw) * wid
    ph = jnp.exp(dh) * hei
    x1 = jnp.clip(pcx - 0.5 * pw, 0.0, _IMG)
    y1 = jnp.clip(pcy - 0.5 * ph, 0.0, _IMG)
    x2 = jnp.clip(pcx + 0.5 * pw, 0.0, _IMG)
    y2 = jnp.clip(pcy + 0.5 * ph, 0.0, _IMG)

    rio = lax.broadcasted_iota(jnp.int32, (_C, _N), 0)

    ws = x2 - x1
    hs = y2 - y1
    valid = (rio > 0) & (p > _SCORE_THRESH) & (ws >= 0.01) & (hs >= 0.01)
    cur0 = jnp.where(valid, p, _NEG)

    x1_ref[...] = x1
    y1_ref[...] = y1
    x2_ref[...] = x2
    y2_ref[...] = y2
    cur_ref[...] = cur0
    cmx_ref[...] = jnp.max(cur0, axis=0, keepdims=True)
    ob_ref[...] = jnp.zeros((_DETS, 4), jnp.float32)
    os_ref[...] = jnp.zeros((_DETS, 1), jnp.float32)
    ol_ref[...] = jnp.zeros((_DETS, 1), jnp.int32)

    lio1 = lax.broadcasted_iota(jnp.int32, (1, _N), 1)
    lio128 = lax.broadcasted_iota(jnp.int32, (1, 128), 1)
    lioc = lax.broadcasted_iota(jnp.int32, (_C, 128), 1)
    rioc = lax.broadcasted_iota(jnp.int32, (_C, 128), 0)
    l4 = lax.broadcasted_iota(jnp.int32, (1, 4), 1)
    big = jnp.int32(1 << 30)

    def cond(st):
        k, alive, pops = st
        return alive & (k < _DETS) & (pops < (1 << 20))

    def body(st):
        k, alive, pops = st
        cm = cmx_ref[...]
        gmax = jnp.max(cm)
        n = jnp.min(jnp.where(cm == gmax, lio1, big))
        nb = pl.multiple_of((n // 128) * 128, 128)
        j = n - nb
        col = cur_ref[:, pl.ds(nb, 128)]
        cmt = jnp.max(jnp.where(lioc == j, col, -jnp.inf))
        stale = cmt < gmax

        def fix(_):
            cmb = cmx_ref[0:1, pl.ds(nb, 128)]
            cmx_ref[0:1, pl.ds(nb, 128)] = jnp.where(lio128 == j, cmt, cmb)
            return k, jnp.bool_(True)

        def keep(_):
            ok = cmt > (_NEG / 2)
            pmc = (lioc == j)
            c = jnp.min(jnp.where(pmc & (col == cmt), rioc, big))
            pm = pmc & (rioc == c)
            cx1 = jnp.max(jnp.where(pm, x1_ref[:, pl.ds(nb, 128)], -jnp.inf))
            cy1 = jnp.max(jnp.where(pm, y1_ref[:, pl.ds(nb, 128)], -jnp.inf))
            cx2 = jnp.max(jnp.where(pm, x2_ref[:, pl.ds(nb, 128)], -jnp.inf))
            cy2 = jnp.max(jnp.where(pm, y2_ref[:, pl.ds(nb, 128)], -jnp.inf))
            rx1 = x1_ref[pl.ds(c, 1), :]
            ry1 = y1_ref[pl.ds(c, 1), :]
            rx2 = x2_ref[pl.ds(c, 1), :]
            ry2 = y2_ref[pl.ds(c, 1), :]
            ix1 = jnp.maximum(cx1, rx1)
            iy1 = jnp.maximum(cy1, ry1)
            ix2 = jnp.minimum(cx2, rx2)
            iy2 = jnp.minimum(cy2, ry2)
            inter = jnp.maximum(ix2 - ix1, 0.0) * jnp.maximum(iy2 - iy1, 0.0)
            a1 = (cx2 - cx1) * (cy2 - cy1)
            a2 = (rx2 - rx1) * (ry2 - ry1)
            iou = inter / (a1 + a2 - inter + 1e-9)
            rcur = cur_ref[pl.ds(c, 1), :]
            newr = jnp.where((iou > _NMS_THRESH) | (lio1 == n), _NEG, rcur)
            cur_ref[pl.ds(c, 1), :] = newr
            okf = jnp.where(ok, 1.0, 0.0)
            row = jnp.where(l4 == 0, cx1,
                  jnp.where(l4 == 1, cy1,
                  jnp.where(l4 == 2, cx2, cy2))) * okf
            ob_ref[pl.ds(k, 1), :] = row
            os_ref[pl.ds(k, 1), :] = jnp.full((1, 1), jnp.where(ok, cmt, 0.0),
                                              dtype=jnp.float32)
            ol_ref[pl.ds(k, 1), :] = jnp.full((1, 1), jnp.where(ok, c, 0),
                                              dtype=jnp.int32)
            return jnp.where(ok, k + 1, k), ok

        k2, alive2 = lax.cond(stale, fix, keep, 0)
        return k2, alive2, pops + 1

    lax.while_loop(cond, body, (jnp.int32(0), jnp.bool_(True), jnp.int32(0)))


def kernel(class_logits, box_regression, proposals):
    lt = class_logits.T
    br = box_regression.reshape(_N, _C, 4)
    dxt = br[:, :, 0].T
    dyt = br[:, :, 1].T
    dwt = br[:, :, 2].T
    dht = br[:, :, 3].T
    pt = proposals.T
    ob, osc, olb = pl.pallas_call(
        _nms_body,
        out_shape=(
            jax.ShapeDtypeStruct((_DETS, 4), jnp.float32),
            jax.ShapeDtypeStruct((_DETS, 1), jnp.float32),
            jax.ShapeDtypeStruct((_DETS, 1), jnp.int32),
        ),
        scratch_shapes=[
            pltpu.VMEM((_C, _N), jnp.float32),
            pltpu.VMEM((1, _N), jnp.float32),
            pltpu.VMEM((_C, _N), jnp.float32),
            pltpu.VMEM((_C, _N), jnp.float32),
            pltpu.VMEM((_C, _N), jnp.float32),
            pltpu.VMEM((_C, _N), jnp.float32),
        ],
        compiler_params=pltpu.CompilerParams(
            vmem_limit_bytes=128 * 1024 * 1024,
        ),
    )(lt, dxt, dyt, dwt, dht, pt)
    return ob, osc.reshape(_DETS), olb.reshape(_DETS)


# speedup vs baseline: 20.2496x; 1.1139x over previous
import jax
import jax.numpy as jnp
from jax import lax
from jax.experimental import pallas as pl
from jax.experimental.pallas import tpu as pltpu
import math

_N = 20000
_NP = 20096
_NB = 157
_C = 21
_IMG = 800.0
_SCORE_THRESH = 0.05
_NMS_THRESH = 0.5
_DETS = 100
_CLIP = math.log(1000.0 / 16.0)
_NEG = -1e10


def _nms_body(lt_ref, dx_ref, dy_ref, dw_ref, dh_ref, pt_ref,
              ob_ref, os_ref, ol_ref,
              cur_ref, x1_ref, y1_ref, x2_ref, y2_ref, a2_ref):
    lt = lt_ref[...]
    m = jnp.max(lt, axis=0, keepdims=True)
    e = jnp.exp(lt - m)
    p = e / jnp.sum(e, axis=0, keepdims=True)

    x1r = pt_ref[0:1, :]
    y1r = pt_ref[1:2, :]
    x2r = pt_ref[2:3, :]
    y2r = pt_ref[3:4, :]
    wid = x2r - x1r
    hei = y2r - y1r
    cx = x1r + 0.5 * wid
    cy = y1r + 0.5 * hei
    dx = dx_ref[...] / 10.0
    dy = dy_ref[...] / 10.0
    dw = jnp.minimum(dw_ref[...] / 5.0, _CLIP)
    dh = jnp.minimum(dh_ref[...] / 5.0, _CLIP)
    pcx = dx * wid + cx
    pcy = dy * hei + cy
    pw = jnp.exp(dw) * wid
    ph = jnp.exp(dh) * hei
    x1 = jnp.clip(pcx - 0.5 * pw, 0.0, _IMG)
    y1 = jnp.clip(pcy - 0.5 * ph, 0.0, _IMG)
    x2 = jnp.clip(pcx + 0.5 * pw, 0.0, _IMG)
    y2 = jnp.clip(pcy + 0.5 * ph, 0.0, _IMG)

    rio = lax.broadcasted_iota(jnp.int32, (_C, _NP), 0)

    ws = x2 - x1
    hs = y2 - y1
    valid = (rio > 0) & (p > _SCORE_THRESH) & (ws >= 0.01) & (hs >= 0.01)
    cur0 = jnp.where(valid, p, _NEG)

    x1_ref[...] = x1
    y1_ref[...] = y1
    x2_ref[...] = x2
    y2_ref[...] = y2
    a2_ref[...] = ws * hs
    cur_ref[...] = cur0
    ob_ref[...] = jnp.zeros((_DETS, 4), jnp.float32)
    os_ref[...] = jnp.zeros((_DETS, 1), jnp.float32)
    ol_ref[...] = jnp.zeros((_DETS, 1), jnp.int32)

    lio1 = lax.broadcasted_iota(jnp.int32, (1, _NP), 1)
    lio128 = lax.broadcasted_iota(jnp.int32, (1, 128), 1)
    lioc = lax.broadcasted_iota(jnp.int32, (_C, 128), 1)
    rioc = lax.broadcasted_iota(jnp.int32, (_C, 128), 0)
    lio256 = lax.broadcasted_iota(jnp.int32, (1, 256), 1)
    l4 = lax.broadcasted_iota(jnp.int32, (1, 4), 1)
    big = jnp.int32(1 << 30)

    def sm_init(i, sm):
        nb = pl.multiple_of(i * 128, 128)
        blk = cur_ref[:, pl.ds(nb, 128)]
        return jnp.where(lio256 == i, jnp.max(blk), sm)

    sm0 = lax.fori_loop(0, _NB, sm_init,
                        jnp.full((1, 256), _NEG, jnp.float32))

    def cond(st):
        k, alive, pops, _ = st
        return alive & (k < _DETS) & (pops < (1 << 20))

    def body(st):
        k, alive, pops, sm = st
        gmax = jnp.max(sm)
        b = jnp.min(jnp.where(sm == gmax, lio256, big))
        nb = pl.multiple_of(b * 128, 128)
        col = cur_ref[:, pl.ds(nb, 128)]
        mrow = jnp.max(col, axis=0, keepdims=True)
        mt = jnp.max(mrow)
        stale = mt < gmax

        def fix(_):
            return k, jnp.bool_(True), jnp.where(lio256 == b, mt, sm)

        def keep(_):
            ok = mt > (_NEG / 2)
            j = jnp.min(jnp.where(mrow == mt, lio128, big))
            pmc = (lioc == j)
            c = jnp.min(jnp.where(pmc & (col == mt), rioc, big))
            pm = pmc & (rioc == c)
            cx1 = jnp.max(jnp.where(pm, x1_ref[:, pl.ds(nb, 128)], -jnp.inf))
            cy1 = jnp.max(jnp.where(pm, y1_ref[:, pl.ds(nb, 128)], -jnp.inf))
            cx2 = jnp.max(jnp.where(pm, x2_ref[:, pl.ds(nb, 128)], -jnp.inf))
            cy2 = jnp.max(jnp.where(pm, y2_ref[:, pl.ds(nb, 128)], -jnp.inf))
            rx1 = x1_ref[pl.ds(c, 1), :]
            ry1 = y1_ref[pl.ds(c, 1), :]
            rx2 = x2_ref[pl.ds(c, 1), :]
            ry2 = y2_ref[pl.ds(c, 1), :]
            ra2 = a2_ref[pl.ds(c, 1), :]
            ix1 = jnp.maximum(cx1, rx1)
            iy1 = jnp.maximum(cy1, ry1)
            ix2 = jnp.minimum(cx2, rx2)
            iy2 = jnp.minimum(cy2, ry2)
            inter = jnp.maximum(ix2 - ix1, 0.0) * jnp.maximum(iy2 - iy1, 0.0)
            a1 = (cx2 - cx1) * (cy2 - cy1)
            union = a1 + ra2 - inter + 1e-9
            supp = inter > 0.5 * union
            n = nb + j
            rcur = cur_ref[pl.ds(c, 1), :]
            newr = jnp.where(supp | (lio1 == n), _NEG, rcur)
            cur_ref[pl.ds(c, 1), :] = newr
            okf = jnp.where(ok, 1.0, 0.0)
            row = jnp.where(l4 == 0, cx1,
                  jnp.where(l4 == 1, cy1,
                  jnp.where(l4 == 2, cx2, cy2))) * okf
            ob_ref[pl.ds(k, 1), :] = row
            os_ref[pl.ds(k, 1), :] = jnp.full((1, 1), jnp.where(ok, mt, 0.0),
                                              dtype=jnp.float32)
            ol_ref[pl.ds(k, 1), :] = jnp.full((1, 1), jnp.where(ok, c, 0),
                                              dtype=jnp.int32)
            blk_after = cur_ref[:, pl.ds(nb, 128)]
            sm2 = jnp.where(lio256 == b, jnp.max(blk_after), sm)
            return jnp.where(ok, k + 1, k), ok, sm2

        k2, alive2, sm3 = lax.cond(stale, fix, keep, 0)
        return k2, alive2, pops + 1, sm3

    lax.while_loop(cond, body,
                   (jnp.int32(0), jnp.bool_(True), jnp.int32(0), sm0))


def kernel(class_logits, box_regression, proposals):
    pad = _NP - _N
    lt = jnp.pad(class_logits.T, ((0, 0), (0, pad)))
    br = box_regression.reshape(_N, _C, 4)
    dxt = jnp.pad(br[:, :, 0].T, ((0, 0), (0, pad)))
    dyt = jnp.pad(br[:, :, 1].T, ((0, 0), (0, pad)))
    dwt = jnp.pad(br[:, :, 2].T, ((0, 0), (0, pad)))
    dht = jnp.pad(br[:, :, 3].T, ((0, 0), (0, pad)))
    pt = jnp.pad(proposals.T, ((0, 0), (0, pad)))
    ob, osc, olb = pl.pallas_call(
        _nms_body,
        out_shape=(
            jax.ShapeDtypeStruct((_DETS, 4), jnp.float32),
            jax.ShapeDtypeStruct((_DETS, 1), jnp.float32),
            jax.ShapeDtypeStruct((_DETS, 1), jnp.int32),
        ),
        scratch_shapes=[
            pltpu.VMEM((_C, _NP), jnp.float32),
            pltpu.VMEM((_C, _NP), jnp.float32),
            pltpu.VMEM((_C, _NP), jnp.float32),
            pltpu.VMEM((_C, _NP), jnp.float32),
            pltpu.VMEM((_C, _NP), jnp.float32),
            pltpu.VMEM((_C, _NP), jnp.float32),
        ],
        compiler_params=pltpu.CompilerParams(
            vmem_limit_bytes=128 * 1024 * 1024,
        ),
    )(lt, dxt, dyt, dwt, dht, pt)
    return ob, osc.reshape(_DETS), olb.reshape(_DETS)


# speedup vs baseline: 26.9941x; 1.3331x over previous
import jax
import jax.numpy as jnp
from jax import lax
from jax.experimental import pallas as pl
from jax.experimental.pallas import tpu as pltpu
import math

_N = 20000
_NP = 20480
_S = 2560
_C = 21
_IMG = 800.0
_SCORE_THRESH = 0.05
_NMS_THRESH = 0.5
_DETS = 100
_CLIP = math.log(1000.0 / 16.0)
_NEG = -1e10


def _nms_body(lt_ref, dx_ref, dy_ref, dw_ref, dh_ref, pt_ref,
              ob_ref, os_ref, ol_ref,
              cur_ref, x1_ref, y1_ref, x2_ref, y2_ref, a2_ref, cmx_ref):
    lt = lt_ref[...]
    m = jnp.max(lt, axis=0, keepdims=True)
    e = jnp.exp(lt - m)
    p = e / jnp.sum(e, axis=0, keepdims=True)

    x1r = pt_ref[0:1, :, :]
    y1r = pt_ref[1:2, :, :]
    x2r = pt_ref[2:3, :, :]
    y2r = pt_ref[3:4, :, :]
    wid = x2r - x1r
    hei = y2r - y1r
    cx = x1r + 0.5 * wid
    cy = y1r + 0.5 * hei
    dx = dx_ref[...] / 10.0
    dy = dy_ref[...] / 10.0
    dw = jnp.minimum(dw_ref[...] / 5.0, _CLIP)
    dh = jnp.minimum(dh_ref[...] / 5.0, _CLIP)
    pcx = dx * wid + cx
    pcy = dy * hei + cy
    pw = jnp.exp(dw) * wid
    ph = jnp.exp(dh) * hei
    x1 = jnp.clip(pcx - 0.5 * pw, 0.0, _IMG)
    y1 = jnp.clip(pcy - 0.5 * ph, 0.0, _IMG)
    x2 = jnp.clip(pcx + 0.5 * pw, 0.0, _IMG)
    y2 = jnp.clip(pcy + 0.5 * ph, 0.0, _IMG)

    rio = lax.broadcasted_iota(jnp.int32, (_C, 8, _S), 0)

    ws = x2 - x1
    hs = y2 - y1
    valid = (rio > 0) & (p > _SCORE_THRESH) & (ws >= 0.01) & (hs >= 0.01)
    cur0 = jnp.where(valid, p, _NEG)

    x1_ref[...] = x1
    y1_ref[...] = y1
    x2_ref[...] = x2
    y2_ref[...] = y2
    a2_ref[...] = ws * hs
    cur_ref[...] = cur0
    cmx_ref[...] = jnp.max(cur0, axis=0, keepdims=True)
    ob_ref[...] = jnp.zeros((_DETS, 4), jnp.float32)
    os_ref[...] = jnp.zeros((_DETS, 1), jnp.float32)
    ol_ref[...] = jnp.zeros((_DETS, 1), jnp.int32)

    ni3 = (lax.broadcasted_iota(jnp.int32, (1, 8, _S), 1) * _S +
           lax.broadcasted_iota(jnp.int32, (1, 8, _S), 2))
    ni_row = ni3
    sio_cb = lax.broadcasted_iota(jnp.int32, (_C, 8, 128), 1)
    lio_cb = lax.broadcasted_iota(jnp.int32, (_C, 8, 128), 2)
    rio_cb = lax.broadcasted_iota(jnp.int32, (_C, 8, 128), 0)
    l4 = lax.broadcasted_iota(jnp.int32, (1, 4), 1)
    big = jnp.int32(1 << 30)

    def cond(st):
        k, alive, pops = st
        return alive & (k < _DETS) & (pops < (1 << 20))

    def body(st):
        k, alive, pops = st
        cm = cmx_ref[...]
        gmax = jnp.max(cm)
        n = jnp.min(jnp.where(cm == gmax, ni3, big))
        s = n // _S
        l = n - s * _S
        g = pl.multiple_of((l // 128) * 128, 128)
        j = l - g
        col8 = cur_ref[:, :, pl.ds(g, 128)]
        pmc = (sio_cb == s) & (lio_cb == j)
        cmt = jnp.max(jnp.where(pmc, col8, -jnp.inf))
        stale = cmt < gmax

        def fix(_):
            cmx_ref[0:1, :, pl.ds(g, 128)] = jnp.max(col8, axis=0,
                                                     keepdims=True)
            return k, jnp.bool_(True)

        def keep(_):
            ok = cmt > (_NEG / 2)
            c = jnp.min(jnp.where(pmc & (col8 == cmt), rio_cb, big))
            pm = pmc & (rio_cb == c)
            cx1 = jnp.max(jnp.where(pm, x1_ref[:, :, pl.ds(g, 128)], -jnp.inf))
            cy1 = jnp.max(jnp.where(pm, y1_ref[:, :, pl.ds(g, 128)], -jnp.inf))
            cx2 = jnp.max(jnp.where(pm, x2_ref[:, :, pl.ds(g, 128)], -jnp.inf))
            cy2 = jnp.max(jnp.where(pm, y2_ref[:, :, pl.ds(g, 128)], -jnp.inf))
            rx1 = x1_ref[pl.ds(c, 1), :, :]
            ry1 = y1_ref[pl.ds(c, 1), :, :]
            rx2 = x2_ref[pl.ds(c, 1), :, :]
            ry2 = y2_ref[pl.ds(c, 1), :, :]
            ra2 = a2_ref[pl.ds(c, 1), :, :]
            ix1 = jnp.maximum(cx1, rx1)
            iy1 = jnp.maximum(cy1, ry1)
            ix2 = jnp.minimum(cx2, rx2)
            iy2 = jnp.minimum(cy2, ry2)
            inter = jnp.maximum(ix2 - ix1, 0.0) * jnp.maximum(iy2 - iy1, 0.0)
            a1 = (cx2 - cx1) * (cy2 - cy1)
            union = a1 + ra2 - inter + 1e-9
            supp = inter > 0.5 * union
            rcur = cur_ref[pl.ds(c, 1), :, :]
            newr = jnp.where(supp | (ni_row == n), _NEG, rcur)
            cur_ref[pl.ds(c, 1), :, :] = newr
            colrep = jnp.where(pm, _NEG, col8)
            cmx_ref[0:1, :, pl.ds(g, 128)] = (
                jnp.max(colrep, axis=0, keepdims=True))
            okf = jnp.where(ok, 1.0, 0.0)
            row = jnp.where(l4 == 0, cx1,
                  jnp.where(l4 == 1, cy1,
                  jnp.where(l4 == 2, cx2, cy2))) * okf
            ob_ref[pl.ds(k, 1), :] = row
            os_ref[pl.ds(k, 1), :] = jnp.full((1, 1), jnp.where(ok, cmt, 0.0),
                                              dtype=jnp.float32)
            ol_ref[pl.ds(k, 1), :] = jnp.full((1, 1), jnp.where(ok, c, 0),
                                              dtype=jnp.int32)
            return jnp.where(ok, k + 1, k), ok

        k2, alive2 = lax.cond(stale, fix, keep, 0)
        return k2, alive2, pops + 1

    lax.while_loop(cond, body, (jnp.int32(0), jnp.bool_(True), jnp.int32(0)))


def kernel(class_logits, box_regression, proposals):
    pad = _NP - _N
    lt = jnp.pad(class_logits.T, ((0, 0), (0, pad))).reshape(_C, 8, _S)
    br = box_regression.reshape(_N, _C, 4)
    dxt = jnp.pad(br[:, :, 0].T, ((0, 0), (0, pad))).reshape(_C, 8, _S)
    dyt = jnp.pad(br[:, :, 1].T, ((0, 0), (0, pad))).reshape(_C, 8, _S)
    dwt = jnp.pad(br[:, :, 2].T, ((0, 0), (0, pad))).reshape(_C, 8, _S)
    dht = jnp.pad(br[:, :, 3].T, ((0, 0), (0, pad))).reshape(_C, 8, _S)
    pt = jnp.pad(proposals.T, ((0, 0), (0, pad))).reshape(4, 8, _S)
    ob, osc, olb = pl.pallas_call(
        _nms_body,
        out_shape=(
            jax.ShapeDtypeStruct((_DETS, 4), jnp.float32),
            jax.ShapeDtypeStruct((_DETS, 1), jnp.float32),
            jax.ShapeDtypeStruct((_DETS, 1), jnp.int32),
        ),
        scratch_shapes=[
            pltpu.VMEM((_C, 8, _S), jnp.float32),
            pltpu.VMEM((_C, 8, _S), jnp.float32),
            pltpu.VMEM((_C, 8, _S), jnp.float32),
            pltpu.VMEM((_C, 8, _S), jnp.float32),
            pltpu.VMEM((_C, 8, _S), jnp.float32),
            pltpu.VMEM((_C, 8, _S), jnp.float32),
            pltpu.VMEM((1, 8, _S), jnp.float32),
        ],
        compiler_params=pltpu.CompilerParams(
            vmem_limit_bytes=128 * 1024 * 1024,
        ),
    )(lt, dxt, dyt, dwt, dht, pt)
    return ob, osc.reshape(_DETS), olb.reshape(_DETS)


# speedup vs baseline: 28.4154x; 1.0526x over previous
import jax
import jax.numpy as jnp
from jax import lax
from jax.experimental import pallas as pl
from jax.experimental.pallas import tpu as pltpu
import math

_N = 20000
_NP = 20480
_S = 2560
_C = 21
_IMG = 800.0
_SCORE_THRESH = 0.05
_NMS_THRESH = 0.5
_DETS = 100
_CLIP = math.log(1000.0 / 16.0)
_NEG = -1e10


def _nms_body(lt_ref, dx_ref, dy_ref, dw_ref, dh_ref, pt_ref,
              ob_ref, os_ref, ol_ref,
              cur_ref, x1_ref, y1_ref, x2_ref, y2_ref, a2_ref, cmx_ref):
    lt = lt_ref[...]
    m = jnp.max(lt, axis=0, keepdims=True)
    e = jnp.exp(lt - m)
    p = e / jnp.sum(e, axis=0, keepdims=True)

    x1r = pt_ref[0:1, :, :]
    y1r = pt_ref[1:2, :, :]
    x2r = pt_ref[2:3, :, :]
    y2r = pt_ref[3:4, :, :]
    wid = x2r - x1r
    hei = y2r - y1r
    cx = x1r + 0.5 * wid
    cy = y1r + 0.5 * hei
    dx = dx_ref[...] / 10.0
    dy = dy_ref[...] / 10.0
    dw = jnp.minimum(dw_ref[...] / 5.0, _CLIP)
    dh = jnp.minimum(dh_ref[...] / 5.0, _CLIP)
    pcx = dx * wid + cx
    pcy = dy * hei + cy
    pw = jnp.exp(dw) * wid
    ph = jnp.exp(dh) * hei
    x1 = jnp.clip(pcx - 0.5 * pw, 0.0, _IMG)
    y1 = jnp.clip(pcy - 0.5 * ph, 0.0, _IMG)
    x2 = jnp.clip(pcx + 0.5 * pw, 0.0, _IMG)
    y2 = jnp.clip(pcy + 0.5 * ph, 0.0, _IMG)

    rio = lax.broadcasted_iota(jnp.int32, (_C, 8, _S), 0)

    ws = x2 - x1
    hs = y2 - y1
    valid = (rio > 0) & (p > _SCORE_THRESH) & (ws >= 0.01) & (hs >= 0.01)
    cur0 = jnp.where(valid, p, _NEG)

    x1_ref[...] = x1
    y1_ref[...] = y1
    x2_ref[...] = x2
    y2_ref[...] = y2
    a2_ref[...] = ws * hs
    cur_ref[...] = cur0
    cmx_ref[...] = jnp.max(cur0, axis=0, keepdims=True)
    ob_ref[...] = jnp.zeros((_DETS, 4), jnp.float32)
    os_ref[...] = jnp.zeros((_DETS, 1), jnp.float32)
    ol_ref[...] = jnp.zeros((_DETS, 1), jnp.int32)

    ni3 = (lax.broadcasted_iota(jnp.int32, (1, 8, _S), 1) * _S +
           lax.broadcasted_iota(jnp.int32, (1, 8, _S), 2))
    ni_row = ni3
    sio_cb = lax.broadcasted_iota(jnp.int32, (_C, 8, 128), 1)
    lio_cb = lax.broadcasted_iota(jnp.int32, (_C, 8, 128), 2)
    rio_cb = lax.broadcasted_iota(jnp.int32, (_C, 8, 128), 0)
    l4 = lax.broadcasted_iota(jnp.int32, (1, 4), 1)
    big = jnp.int32(1 << 30)

    def cond(st):
        k, alive, pops = st
        return alive & (k < _DETS) & (pops < (1 << 20))

    def body(st):
        k, alive, pops = st
        cm = cmx_ref[...]
        gmax = jnp.max(cm)
        n = jnp.min(jnp.where(cm == gmax, ni3, big))
        s = n // _S
        l = n - s * _S
        g = pl.multiple_of((l // 128) * 128, 128)
        j = l - g
        col8 = cur_ref[:, :, pl.ds(g, 128)]
        pmc = (sio_cb == s) & (lio_cb == j)
        cmt = gmax
        ok = cmt > (_NEG / 2)
        c = jnp.min(jnp.where(pmc & (col8 == cmt), rio_cb, big))
        pm = pmc & (rio_cb == c)
        cx1 = jnp.max(jnp.where(pm, x1_ref[:, :, pl.ds(g, 128)], -jnp.inf))
        cy1 = jnp.max(jnp.where(pm, y1_ref[:, :, pl.ds(g, 128)], -jnp.inf))
        cx2 = jnp.max(jnp.where(pm, x2_ref[:, :, pl.ds(g, 128)], -jnp.inf))
        cy2 = jnp.max(jnp.where(pm, y2_ref[:, :, pl.ds(g, 128)], -jnp.inf))
        rx1 = x1_ref[pl.ds(c, 1), :, :]
        ry1 = y1_ref[pl.ds(c, 1), :, :]
        rx2 = x2_ref[pl.ds(c, 1), :, :]
        ry2 = y2_ref[pl.ds(c, 1), :, :]
        ra2 = a2_ref[pl.ds(c, 1), :, :]
        ix1 = jnp.maximum(cx1, rx1)
        iy1 = jnp.maximum(cy1, ry1)
        ix2 = jnp.minimum(cx2, rx2)
        iy2 = jnp.minimum(cy2, ry2)
        inter = jnp.maximum(ix2 - ix1, 0.0) * jnp.maximum(iy2 - iy1, 0.0)
        a1 = (cx2 - cx1) * (cy2 - cy1)
        union = a1 + ra2 - inter + 1e-9
        supp = inter > 0.5 * union
        rcur = cur_ref[pl.ds(c, 1), :, :]
        newr = jnp.where(supp | (ni_row == n), _NEG, rcur)
        cur_ref[pl.ds(c, 1), :, :] = newr
        cmx_ref[...] = jnp.max(cur_ref[...], axis=0, keepdims=True)
        okf = jnp.where(ok, 1.0, 0.0)
        row = jnp.where(l4 == 0, cx1,
              jnp.where(l4 == 1, cy1,
              jnp.where(l4 == 2, cx2, cy2))) * okf
        ob_ref[pl.ds(k, 1), :] = row
        os_ref[pl.ds(k, 1), :] = jnp.full((1, 1), jnp.where(ok, cmt, 0.0),
                                          dtype=jnp.float32)
        ol_ref[pl.ds(k, 1), :] = jnp.full((1, 1), jnp.where(ok, c, 0),
                                          dtype=jnp.int32)
        return jnp.where(ok, k + 1, k), ok, pops + 1

    lax.while_loop(cond, body, (jnp.int32(0), jnp.bool_(True), jnp.int32(0)))


def kernel(class_logits, box_regression, proposals):
    pad = _NP - _N
    lt = jnp.pad(class_logits.T, ((0, 0), (0, pad))).reshape(_C, 8, _S)
    br = box_regression.reshape(_N, _C, 4)
    dxt = jnp.pad(br[:, :, 0].T, ((0, 0), (0, pad))).reshape(_C, 8, _S)
    dyt = jnp.pad(br[:, :, 1].T, ((0, 0), (0, pad))).reshape(_C, 8, _S)
    dwt = jnp.pad(br[:, :, 2].T, ((0, 0), (0, pad))).reshape(_C, 8, _S)
    dht = jnp.pad(br[:, :, 3].T, ((0, 0), (0, pad))).reshape(_C, 8, _S)
    pt = jnp.pad(proposals.T, ((0, 0), (0, pad))).reshape(4, 8, _S)
    ob, osc, olb = pl.pallas_call(
        _nms_body,
        out_shape=(
            jax.ShapeDtypeStruct((_DETS, 4), jnp.float32),
            jax.ShapeDtypeStruct((_DETS, 1), jnp.float32),
            jax.ShapeDtypeStruct((_DETS, 1), jnp.int32),
        ),
        scratch_shapes=[
            pltpu.VMEM((_C, 8, _S), jnp.float32),
            pltpu.VMEM((_C, 8, _S), jnp.float32),
            pltpu.VMEM((_C, 8, _S), jnp.float32),
            pltpu.VMEM((_C, 8, _S), jnp.float32),
            pltpu.VMEM((_C, 8, _S), jnp.float32),
            pltpu.VMEM((_C, 8, _S), jnp.float32),
            pltpu.VMEM((1, 8, _S), jnp.float32),
        ],
        compiler_params=pltpu.CompilerParams(
            vmem_limit_bytes=128 * 1024 * 1024,
        ),
    )(lt, dxt, dyt, dwt, dht, pt)
    return ob, osc.reshape(_DETS), olb.reshape(_DETS)


# speedup vs baseline: 28.4198x; 1.0002x over previous
import jax
import jax.numpy as jnp
from jax import lax
from jax.experimental import pallas as pl
from jax.experimental.pallas import tpu as pltpu
import math

_N = 20000
_NP = 20480
_S = 2560
_C = 21
_IMG = 800.0
_SCORE_THRESH = 0.05
_NMS_THRESH = 0.5
_DETS = 100
_CLIP = math.log(1000.0 / 16.0)
_NEG = -1e10


def _nms_body(lt_ref, dx_ref, dy_ref, dw_ref, dh_ref, pt_ref,
              ob_ref, os_ref, ol_ref,
              cur_ref, x1_ref, y1_ref, x2_ref, y2_ref, a2_ref, cmx_ref):
    lt = lt_ref[...]
    m = jnp.max(lt, axis=0, keepdims=True)
    e = jnp.exp(lt - m)
    p = e * (1.0 / jnp.sum(e, axis=0, keepdims=True))

    x1r = pt_ref[0:1, :, :]
    y1r = pt_ref[1:2, :, :]
    x2r = pt_ref[2:3, :, :]
    y2r = pt_ref[3:4, :, :]
    wid = x2r - x1r
    hei = y2r - y1r
    cx = x1r + 0.5 * wid
    cy = y1r + 0.5 * hei
    dx = dx_ref[...] * 0.1
    dy = dy_ref[...] * 0.1
    dw = jnp.minimum(dw_ref[...] * 0.2, _CLIP)
    dh = jnp.minimum(dh_ref[...] * 0.2, _CLIP)
    pcx = dx * wid + cx
    pcy = dy * hei + cy
    pw = jnp.exp(dw) * wid
    ph = jnp.exp(dh) * hei
    x1 = jnp.clip(pcx - 0.5 * pw, 0.0, _IMG)
    y1 = jnp.clip(pcy - 0.5 * ph, 0.0, _IMG)
    x2 = jnp.clip(pcx + 0.5 * pw, 0.0, _IMG)
    y2 = jnp.clip(pcy + 0.5 * ph, 0.0, _IMG)

    rio = lax.broadcasted_iota(jnp.int32, (_C, 8, _S), 0)

    ws = x2 - x1
    hs = y2 - y1
    valid = (rio > 0) & (p > _SCORE_THRESH) & (ws >= 0.01) & (hs >= 0.01)
    cur0 = jnp.where(valid, p, _NEG)

    x1_ref[...] = x1
    y1_ref[...] = y1
    x2_ref[...] = x2
    y2_ref[...] = y2
    a2_ref[...] = ws * hs
    cur_ref[...] = cur0
    cmx_ref[...] = jnp.max(cur0, axis=0, keepdims=True)
    ob_ref[...] = jnp.zeros((_DETS, 4), jnp.float32)
    os_ref[...] = jnp.zeros((_DETS, 1), jnp.float32)
    ol_ref[...] = jnp.zeros((_DETS, 1), jnp.int32)

    ni3 = (lax.broadcasted_iota(jnp.int32, (1, 8, _S), 1) * _S +
           lax.broadcasted_iota(jnp.int32, (1, 8, _S), 2))
    ni_row = ni3
    sio_cb = lax.broadcasted_iota(jnp.int32, (_C, 8, 128), 1)
    lio_cb = lax.broadcasted_iota(jnp.int32, (_C, 8, 128), 2)
    rio_cb = lax.broadcasted_iota(jnp.int32, (_C, 8, 128), 0)
    l4 = lax.broadcasted_iota(jnp.int32, (1, 4), 1)
    big = jnp.int32(1 << 30)

    def cond(st):
        k, alive = st
        return alive & (k < _DETS)

    def body(st):
        k, alive = st
        cm = cmx_ref[...]
        gmax = jnp.max(cm)
        n = jnp.min(jnp.where(cm == gmax, ni3, big))
        s = n // _S
        l = n - s * _S
        g = pl.multiple_of((l // 128) * 128, 128)
        j = l - g
        col8 = cur_ref[:, :, pl.ds(g, 128)]
        pmc = (sio_cb == s) & (lio_cb == j)
        cmt = gmax
        ok = cmt > (_NEG / 2)
        c = jnp.min(jnp.where(pmc & (col8 == cmt), rio_cb, big))
        pm = pmc & (rio_cb == c)
        cx1 = jnp.max(jnp.where(pm, x1_ref[:, :, pl.ds(g, 128)], -jnp.inf))
        cy1 = jnp.max(jnp.where(pm, y1_ref[:, :, pl.ds(g, 128)], -jnp.inf))
        cx2 = jnp.max(jnp.where(pm, x2_ref[:, :, pl.ds(g, 128)], -jnp.inf))
        cy2 = jnp.max(jnp.where(pm, y2_ref[:, :, pl.ds(g, 128)], -jnp.inf))
        rx1 = x1_ref[pl.ds(c, 1), :, :]
        ry1 = y1_ref[pl.ds(c, 1), :, :]
        rx2 = x2_ref[pl.ds(c, 1), :, :]
        ry2 = y2_ref[pl.ds(c, 1), :, :]
        ra2 = a2_ref[pl.ds(c, 1), :, :]
        ix1 = jnp.maximum(cx1, rx1)
        iy1 = jnp.maximum(cy1, ry1)
        ix2 = jnp.minimum(cx2, rx2)
        iy2 = jnp.minimum(cy2, ry2)
        inter = jnp.maximum(ix2 - ix1, 0.0) * jnp.maximum(iy2 - iy1, 0.0)
        a1 = (cx2 - cx1) * (cy2 - cy1)
        union = a1 + ra2 - inter + 1e-9
        supp = inter > 0.5 * union
        rcur = cur_ref[pl.ds(c, 1), :, :]
        newr = jnp.where(supp | (ni_row == n), _NEG, rcur)
        cur_ref[pl.ds(c, 1), :, :] = newr
        cmx_ref[...] = jnp.max(cur_ref[...], axis=0, keepdims=True)
        okf = jnp.where(ok, 1.0, 0.0)
        row = jnp.where(l4 == 0, cx1,
              jnp.where(l4 == 1, cy1,
              jnp.where(l4 == 2, cx2, cy2))) * okf
        ob_ref[pl.ds(k, 1), :] = row
        os_ref[pl.ds(k, 1), :] = jnp.full((1, 1), jnp.where(ok, cmt, 0.0),
                                          dtype=jnp.float32)
        ol_ref[pl.ds(k, 1), :] = jnp.full((1, 1), jnp.where(ok, c, 0),
                                          dtype=jnp.int32)
        return jnp.where(ok, k + 1, k), ok

    lax.while_loop(cond, body, (jnp.int32(0), jnp.bool_(True)))


def kernel(class_logits, box_regression, proposals):
    pad = _NP - _N
    lt = jnp.pad(class_logits.T, ((0, 0), (0, pad))).reshape(_C, 8, _S)
    br = box_regression.reshape(_N, _C, 4)
    dxt = jnp.pad(br[:, :, 0].T, ((0, 0), (0, pad))).reshape(_C, 8, _S)
    dyt = jnp.pad(br[:, :, 1].T, ((0, 0), (0, pad))).reshape(_C, 8, _S)
    dwt = jnp.pad(br[:, :, 2].T, ((0, 0), (0, pad))).reshape(_C, 8, _S)
    dht = jnp.pad(br[:, :, 3].T, ((0, 0), (0, pad))).reshape(_C, 8, _S)
    pt = jnp.pad(proposals.T, ((0, 0), (0, pad))).reshape(4, 8, _S)
    ob, osc, olb = pl.pallas_call(
        _nms_body,
        out_shape=(
            jax.ShapeDtypeStruct((_DETS, 4), jnp.float32),
            jax.ShapeDtypeStruct((_DETS, 1), jnp.float32),
            jax.ShapeDtypeStruct((_DETS, 1), jnp.int32),
        ),
        scratch_shapes=[
            pltpu.VMEM((_C, 8, _S), jnp.float32),
            pltpu.VMEM((_C, 8, _S), jnp.float32),
            pltpu.VMEM((_C, 8, _S), jnp.float32),
            pltpu.VMEM((_C, 8, _S), jnp.float32),
            pltpu.VMEM((_C, 8, _S), jnp.float32),
            pltpu.VMEM((_C, 8, _S), jnp.float32),
            pltpu.VMEM((1, 8, _S), jnp.float32),
        ],
        compiler_params=pltpu.CompilerParams(
            vmem_limit_bytes=128 * 1024 * 1024,
        ),
    )(lt, dxt, dyt, dwt, dht, pt)
    return ob, osc.reshape(_DETS), olb.reshape(_DETS)
